# baseline (speedup 1.0000x reference)
import functools

import jax
import jax.numpy as jnp
from jax import lax
from jax.experimental import pallas as pl
from jax.experimental.pallas import tpu as pltpu
from jax.experimental.pallas import tpu_sc as plsc

_IDX = 13
_B, _F, _D = 16384, 26, 64
_NW = 32
_BPW = _B // _NW
_NBUF = 2
_CHUNK = 128
_NCHUNK = _BPW // _CHUNK


def _body(in_hbm, out_hbm, buf_v, in_sems, out_sems):
    wid = lax.axis_index("s") * 2 + lax.axis_index("c")
    base = wid * _BPW
    col = _IDX * _D

    def in_copy(i):
        b = i % _NBUF
        return pltpu.make_async_copy(
            in_hbm.at[pl.ds(base + i * _CHUNK, _CHUNK), pl.ds(col, _D)],
            buf_v.at[b],
            in_sems.at[b],
        )

    def out_copy(i):
        b = i % _NBUF
        return pltpu.make_async_copy(
            buf_v.at[b],
            out_hbm.at[pl.ds(base + i * _CHUNK, _CHUNK)],
            out_sems.at[b],
        )

    for b in range(_NBUF):
        in_copy(b).start()
    for i in range(_NCHUNK):
        in_copy(i).wait()
        out_copy(i).start()
        nxt = i + _NBUF
        if nxt < _NCHUNK:
            out_copy(i).wait()
            in_copy(nxt).start()
    for i in range(max(0, _NCHUNK - _NBUF), _NCHUNK):
        out_copy(i).wait()


def kernel(inputs):
    flat = inputs.reshape(_B, _F * _D)
    mesh = plsc.VectorSubcoreMesh(core_axis_name="c", subcore_axis_name="s")

    run = functools.partial(
        pl.kernel,
        mesh=mesh,
        out_type=jax.ShapeDtypeStruct((_B, _D), jnp.float32),
        scratch_types=[
            pltpu.VMEM((_NBUF, _CHUNK, _D), jnp.float32),
            pltpu.SemaphoreType.DMA((_NBUF,)),
            pltpu.SemaphoreType.DMA((_NBUF,)),
        ],
        compiler_params=pltpu.CompilerParams(use_tc_tiling_on_sc=False),
    )(_body)
    return run(flat)


# speedup vs baseline: 1.4951x; 1.4951x over previous
import functools

import jax
import jax.numpy as jnp
from jax import lax
from jax.experimental import pallas as pl
from jax.experimental.pallas import tpu as pltpu
from jax.experimental.pallas import tpu_sc as plsc

_IDX = 13
_B, _F, _D = 16384, 26, 64
_NW = 32
_CPW = _B // _NW


def _body(in_hbm, out_hbm):
    wid = lax.axis_index("s") * 2 + lax.axis_index("c")
    base = wid * _CPW
    pltpu.sync_copy(
        in_hbm.at[_IDX, :, pl.ds(base, _CPW)],
        out_hbm.at[:, pl.ds(base, _CPW)],
    )


def kernel(inputs):
    plane_major = jnp.transpose(inputs, (1, 2, 0))
    mesh = plsc.VectorSubcoreMesh(core_axis_name="c", subcore_axis_name="s")
    run = functools.partial(
        pl.kernel,
        mesh=mesh,
        out_type=jax.ShapeDtypeStruct((_D, _B), jnp.float32),
    )(_body)
    return run(plane_major).T


# speedup vs baseline: 9.4620x; 6.3285x over previous
import functools

import jax
import jax.numpy as jnp
from jax import lax
from jax.experimental import pallas as pl
from jax.experimental.pallas import tpu as pltpu
from jax.experimental.pallas import tpu_sc as plsc

_IDX = 13
_B, _F, _D = 16384, 26, 64
_NW = 32
_CPW = _B // _NW
_NBUF = 2
_CHUNK = 128
_NCHUNK = _CPW // _CHUNK


def _body(in_hbm, out_hbm, buf_v, in_sems, out_sems):
    wid = lax.axis_index("s") * 2 + lax.axis_index("c")
    base = wid * _CPW

    def in_copy(i):
        b = i % _NBUF
        return pltpu.make_async_copy(
            in_hbm.at[_IDX, :, pl.ds(base + i * _CHUNK, _CHUNK)],
            buf_v.at[b],
            in_sems.at[b],
        )

    def out_copy(i):
        b = i % _NBUF
        return pltpu.make_async_copy(
            buf_v.at[b],
            out_hbm.at[:, pl.ds(base + i * _CHUNK, _CHUNK)],
            out_sems.at[b],
        )

    for b in range(_NBUF):
        in_copy(b).start()
    for i in range(_NCHUNK):
        in_copy(i).wait()
        out_copy(i).start()
        nxt = i + _NBUF
        if nxt < _NCHUNK:
            out_copy(i).wait()
            in_copy(nxt).start()
    for i in range(max(0, _NCHUNK - _NBUF), _NCHUNK):
        out_copy(i).wait()


def kernel(inputs):
    plane_major = jnp.transpose(inputs, (1, 2, 0))
    mesh = plsc.VectorSubcoreMesh(core_axis_name="c", subcore_axis_name="s")
    run = functools.partial(
        pl.kernel,
        mesh=mesh,
        out_type=jax.ShapeDtypeStruct((_D, _B), jnp.float32),
        scratch_types=[
            pltpu.VMEM((_NBUF, _D, _CHUNK), jnp.float32),
            pltpu.SemaphoreType.DMA((_NBUF,)),
            pltpu.SemaphoreType.DMA((_NBUF,)),
        ],
    )(_body)
    return run(plane_major).T


# speedup vs baseline: 9.7777x; 1.0334x over previous
import functools

import jax
import jax.numpy as jnp
from jax import lax
from jax.experimental import pallas as pl
from jax.experimental.pallas import tpu as pltpu
from jax.experimental.pallas import tpu_sc as plsc

_IDX = 13
_B, _F, _D = 16384, 26, 64
_NW = 32
_CPW = _B // _NW
_NBUF = 2
_CHUNK = 128
_NCHUNK = _CPW // _CHUNK


def _body(in_hbm, out_hbm, buf_v):
    wid = lax.axis_index("s") * 2 + lax.axis_index("c")
    base = wid * _CPW
    pltpu.sync_copy(in_hbm.at[_IDX, :, pl.ds(base, _CPW)], buf_v)
    pltpu.sync_copy(buf_v, out_hbm.at[:, pl.ds(base, _CPW)])


def kernel(inputs):
    plane_major = jnp.transpose(inputs, (1, 2, 0))
    mesh = plsc.VectorSubcoreMesh(core_axis_name="c", subcore_axis_name="s")
    run = functools.partial(
        pl.kernel,
        mesh=mesh,
        out_type=jax.ShapeDtypeStruct((_D, _B), jnp.float32),
        scratch_types=[
            pltpu.VMEM((_D, _CPW), jnp.float32),
        ],
        compiler_params=pltpu.CompilerParams(
            skip_device_barrier=True,
            disable_bounds_checks=True,
            disable_semaphore_checks=True,
        ),
    )(_body)
    return run(plane_major).T
